# baseline (speedup 1.0000x reference)
import jax
import jax.numpy as jnp
from jax.experimental import pallas as pl
from jax.experimental.pallas import tpu as pltpu

_E = 800000
_N = 50000
_H = 16
_ER = 6250


def _tri_body(edges_ref, s_ref, r_ref, out_ref):
    out_ref[...] = jnp.where(s_ref[...] >= r_ref[...], edges_ref[...], 0.0)


def kernel(nodes, edges, receivers, senders, bi_edges_indx, lhs_nodes, lhs_edges,
           lhs_receivers, lhs_senders, node_enc_W, node_enc_b, edge_enc_W,
           edge_enc_b, mp_edge_W, mp_edge_b, mp_node_W, mp_node_b, edge_dec_W,
           edge_dec_b, w, b):
    edges1 = edges[:, 0]
    norm = jnp.abs(edges1).max()
    e = edges1 / norm

    n0 = jnp.tanh(nodes * node_enc_W + node_enc_b)
    ef0 = jnp.tanh(e[:, None] * edge_enc_W + edge_enc_b)

    sn = n0[senders]
    rn = n0[receivers]
    ef1 = jnp.tanh(jnp.concatenate([sn, rn, ef0], axis=-1) @ mp_edge_W + mp_edge_b)

    a = bi_edges_indx[:, 0]
    bcol = bi_edges_indx[:, 1]
    avg = 0.5 * (ef1[a] + ef1[bcol])
    ef2 = ef1.at[a].set(avg)
    ef2 = ef2.at[bcol].set(avg)

    dec = ef2 @ edge_dec_W + edge_dec_b
    alpha = w @ dec + b

    tri = pl.pallas_call(
        _tri_body,
        out_shape=jax.ShapeDtypeStruct((_ER, 128), jnp.float32),
    )(edges1.reshape(_ER, 128),
      senders.reshape(_ER, 128),
      receivers.reshape(_ER, 128))
    tri_data = tri.reshape(_E)

    diag_data = jnp.broadcast_to(alpha, (_N,))
    data = jnp.concatenate([tri_data, diag_data], axis=0)
    ar = jnp.arange(_N, dtype=jnp.int32)
    indices = jnp.concatenate(
        [jnp.stack([senders, receivers], axis=1), jnp.stack([ar, ar], axis=1)],
        axis=0)
    return data, indices


# speedup vs baseline: 1.0258x; 1.0258x over previous
import jax
import jax.numpy as jnp
from jax.experimental import pallas as pl
from jax.experimental.pallas import tpu as pltpu

_E = 800000
_N = 50000
_H = 16
_ER = 6250
_BE1 = 625
_BN = 2000
_BE2 = 4000


def _norm_tri_body(edges_ref, s_ref, r_ref, norm_ref, tri_ref):
    tri_ref[...] = jnp.where(s_ref[...] >= r_ref[...], edges_ref[...], 0.0)
    norm_ref[...] = jnp.max(jnp.abs(edges_ref[...])).reshape(1, 1)


def _pq_body(nodes_ref, nW_ref, nb_ref, Ws_ref, Wr_ref, p_ref, q_ref):
    n0 = jnp.tanh(nodes_ref[...] * nW_ref[...] + nb_ref[...])
    p_ref[...] = jnp.dot(n0, Ws_ref[...], preferred_element_type=jnp.float32)
    q_ref[...] = jnp.dot(n0, Wr_ref[...], preferred_element_type=jnp.float32)


def _edge_body(edges_ref, sp_ref, sq_ref, c_ref, w_ref, norm_ref, eW_ref,
               eb_ref, We3_ref, be_ref, s0_ref, sw_ref):
    i = pl.program_id(0)
    e = edges_ref[...] / norm_ref[...]
    ef0 = jnp.tanh(e * eW_ref[...] + eb_ref[...])
    r = jnp.dot(ef0, We3_ref[...], preferred_element_type=jnp.float32) + be_ref[...]
    ef1 = jnp.tanh(sp_ref[...] + sq_ref[...] + r)
    contrib = jnp.sum(c_ref[...] * ef1, axis=0, keepdims=True)
    wsum = jnp.sum(w_ref[...])

    @pl.when(i == 0)
    def _():
        s0_ref[...] = contrib
        sw_ref[...] = wsum.reshape(1, 1)

    @pl.when(i > 0)
    def _():
        s0_ref[...] = s0_ref[...] + contrib
        sw_ref[...] = sw_ref[...] + wsum.reshape(1, 1)


def kernel(nodes, edges, receivers, senders, bi_edges_indx, lhs_nodes, lhs_edges,
           lhs_receivers, lhs_senders, node_enc_W, node_enc_b, edge_enc_W,
           edge_enc_b, mp_edge_W, mp_edge_b, mp_node_W, mp_node_b, edge_dec_W,
           edge_dec_b, w, b):
    edges1 = edges[:, 0]

    norm2, tri2 = pl.pallas_call(
        _norm_tri_body,
        out_shape=[
            jax.ShapeDtypeStruct((1, 1), jnp.float32),
            jax.ShapeDtypeStruct((_ER, 128), jnp.float32),
        ],
    )(edges1.reshape(_ER, 128), senders.reshape(_ER, 128),
      receivers.reshape(_ER, 128))
    tri_data = tri2.reshape(_E)

    Ws = mp_edge_W[0:_H]
    Wr = mp_edge_W[_H:2 * _H]
    We3 = mp_edge_W[2 * _H:3 * _H]
    P, Q = pl.pallas_call(
        _pq_body,
        grid=(_N // _BN,),
        in_specs=[
            pl.BlockSpec((_BN, 1), lambda i: (i, 0)),
            pl.BlockSpec((1, _H), lambda i: (0, 0)),
            pl.BlockSpec((1, _H), lambda i: (0, 0)),
            pl.BlockSpec((_H, _H), lambda i: (0, 0)),
            pl.BlockSpec((_H, _H), lambda i: (0, 0)),
        ],
        out_specs=[
            pl.BlockSpec((_BN, _H), lambda i: (i, 0)),
            pl.BlockSpec((_BN, _H), lambda i: (i, 0)),
        ],
        out_shape=[
            jax.ShapeDtypeStruct((_N, _H), jnp.float32),
            jax.ShapeDtypeStruct((_N, _H), jnp.float32),
        ],
    )(nodes, node_enc_W, node_enc_b.reshape(1, _H), Ws, Wr)

    SP = P[senders]
    SQ = Q[receivers]

    a = bi_edges_indx[:, 0]
    bcol = bi_edges_indx[:, 1]
    pair_ids = jnp.arange(_E // 2, dtype=jnp.int32)
    win = jnp.full((_E,), -1, jnp.int32).at[a].set(pair_ids).at[bcol].set(pair_ids)
    touched = win >= 0
    Wacc = jnp.zeros((_E // 2,), jnp.float32).at[
        jnp.where(touched, win, 0)].add(jnp.where(touched, w, 0.0))
    G = jnp.zeros((_E,), jnp.float32).at[a].add(Wacc).at[bcol].add(Wacc)
    c = jnp.where(touched, 0.0, w) + 0.5 * G

    S0, sw = pl.pallas_call(
        _edge_body,
        grid=(_E // _BE2,),
        in_specs=[
            pl.BlockSpec((_BE2, 1), lambda i: (i, 0)),
            pl.BlockSpec((_BE2, _H), lambda i: (i, 0)),
            pl.BlockSpec((_BE2, _H), lambda i: (i, 0)),
            pl.BlockSpec((_BE2, 1), lambda i: (i, 0)),
            pl.BlockSpec((_BE2, 1), lambda i: (i, 0)),
            pl.BlockSpec((1, 1), lambda i: (0, 0)),
            pl.BlockSpec((1, _H), lambda i: (0, 0)),
            pl.BlockSpec((1, _H), lambda i: (0, 0)),
            pl.BlockSpec((_H, _H), lambda i: (0, 0)),
            pl.BlockSpec((1, _H), lambda i: (0, 0)),
        ],
        out_specs=[
            pl.BlockSpec((1, _H), lambda i: (0, 0)),
            pl.BlockSpec((1, 1), lambda i: (0, 0)),
        ],
        out_shape=[
            jax.ShapeDtypeStruct((1, _H), jnp.float32),
            jax.ShapeDtypeStruct((1, 1), jnp.float32),
        ],
    )(edges, SP, SQ, c.reshape(_E, 1), w.reshape(_E, 1), norm2,
      edge_enc_W, edge_enc_b.reshape(1, _H), We3, mp_edge_b.reshape(1, _H))

    alpha = (S0 @ edge_dec_W)[0] + edge_dec_b * sw[0, 0] + b

    diag_data = jnp.broadcast_to(alpha, (_N,))
    data = jnp.concatenate([tri_data, diag_data], axis=0)
    ar = jnp.arange(_N, dtype=jnp.int32)
    indices = jnp.concatenate(
        [jnp.stack([senders, receivers], axis=1), jnp.stack([ar, ar], axis=1)],
        axis=0)
    return data, indices


# speedup vs baseline: 1.3665x; 1.3322x over previous
import functools

import jax
import jax.numpy as jnp
from jax import lax
from jax.experimental import pallas as pl
from jax.experimental.pallas import tpu as pltpu
from jax.experimental.pallas import tpu_sc as plsc

_E = 800000
_N = 50000
_H = 16
_ER = 6250
_BE1 = 625
_BN = 2000
_BE2 = 4000


def _norm_tri_body(edges_ref, s_ref, r_ref, norm_ref, tri_ref):
    tri_ref[...] = jnp.where(s_ref[...] >= r_ref[...], edges_ref[...], 0.0)
    norm_ref[...] = jnp.max(jnp.abs(edges_ref[...])).reshape(1, 1)


def _pq_body(nodes_ref, nW_ref, nb_ref, Ws_ref, Wr_ref, p_ref, q_ref):
    n0 = jnp.tanh(nodes_ref[...] * nW_ref[...] + nb_ref[...])
    p_ref[...] = jnp.dot(n0, Ws_ref[...], preferred_element_type=jnp.float32)
    q_ref[...] = jnp.dot(n0, Wr_ref[...], preferred_element_type=jnp.float32)


def _edge_body(edges_ref, sp_ref, sq_ref, c_ref, w_ref, norm_ref, eW_ref,
               eb_ref, We3_ref, be_ref, s0_ref, sw_ref):
    i = pl.program_id(0)
    e = edges_ref[...] / norm_ref[...]
    ef0 = jnp.tanh(e * eW_ref[...] + eb_ref[...])
    r = jnp.dot(ef0, We3_ref[...], preferred_element_type=jnp.float32) + be_ref[...]
    ef1 = jnp.tanh(sp_ref[...] + sq_ref[...] + r)
    contrib = jnp.sum(c_ref[...] * ef1, axis=0, keepdims=True)
    wsum = jnp.sum(w_ref[...])

    @pl.when(i == 0)
    def _():
        s0_ref[...] = contrib
        sw_ref[...] = wsum.reshape(1, 1)

    @pl.when(i > 0)
    def _():
        s0_ref[...] = s0_ref[...] + contrib
        sw_ref[...] = sw_ref[...] + wsum.reshape(1, 1)


_NC = 2
_NS = 16
_NW = _NC * _NS
_EPW = _E // _NW
_GCH = 1000
_GIT = _EPW // _GCH


def _gather_body(p_hbm, q_hbm, s_hbm, r_hbm, sp_hbm, sq_hbm,
                 sidx_v, ridx_v, prow_v, qrow_v, sem1, sem2):
    wid = lax.axis_index("s") * _NC + lax.axis_index("c")
    base = wid * _EPW

    def step(t, carry):
        off = base + t * _GCH
        pltpu.sync_copy(s_hbm.at[pl.ds(off, _GCH)], sidx_v)
        pltpu.sync_copy(r_hbm.at[pl.ds(off, _GCH)], ridx_v)
        cp1 = pltpu.async_copy(p_hbm.at[sidx_v], prow_v, sem1)
        cp2 = pltpu.async_copy(q_hbm.at[ridx_v], qrow_v, sem2)
        cp1.wait()
        cp2.wait()
        pltpu.sync_copy(prow_v, sp_hbm.at[pl.ds(off, _GCH)])
        pltpu.sync_copy(qrow_v, sq_hbm.at[pl.ds(off, _GCH)])
        return carry

    lax.fori_loop(0, _GIT, step, 0)


def _sc_gather(P, Q, senders, receivers):
    mesh = plsc.VectorSubcoreMesh(core_axis_name="c", subcore_axis_name="s",
                                  num_cores=_NC, num_subcores=_NS)
    f = pl.kernel(
        _gather_body,
        out_type=[
            jax.ShapeDtypeStruct((_E, _H), jnp.float32),
            jax.ShapeDtypeStruct((_E, _H), jnp.float32),
        ],
        mesh=mesh,
        scratch_types=[
            pltpu.VMEM((_GCH,), jnp.int32),
            pltpu.VMEM((_GCH,), jnp.int32),
            pltpu.VMEM((_GCH, _H), jnp.float32),
            pltpu.VMEM((_GCH, _H), jnp.float32),
            pltpu.SemaphoreType.DMA,
            pltpu.SemaphoreType.DMA,
        ],
        compiler_params=pltpu.CompilerParams(use_tc_tiling_on_sc=False),
    )
    return f(P, Q, senders, receivers)


def kernel(nodes, edges, receivers, senders, bi_edges_indx, lhs_nodes, lhs_edges,
           lhs_receivers, lhs_senders, node_enc_W, node_enc_b, edge_enc_W,
           edge_enc_b, mp_edge_W, mp_edge_b, mp_node_W, mp_node_b, edge_dec_W,
           edge_dec_b, w, b):
    edges1 = edges[:, 0]

    norm2, tri2 = pl.pallas_call(
        _norm_tri_body,
        out_shape=[
            jax.ShapeDtypeStruct((1, 1), jnp.float32),
            jax.ShapeDtypeStruct((_ER, 128), jnp.float32),
        ],
    )(edges1.reshape(_ER, 128), senders.reshape(_ER, 128),
      receivers.reshape(_ER, 128))
    tri_data = tri2.reshape(_E)

    Ws = mp_edge_W[0:_H]
    Wr = mp_edge_W[_H:2 * _H]
    We3 = mp_edge_W[2 * _H:3 * _H]
    P, Q = pl.pallas_call(
        _pq_body,
        grid=(_N // _BN,),
        in_specs=[
            pl.BlockSpec((_BN, 1), lambda i: (i, 0)),
            pl.BlockSpec((1, _H), lambda i: (0, 0)),
            pl.BlockSpec((1, _H), lambda i: (0, 0)),
            pl.BlockSpec((_H, _H), lambda i: (0, 0)),
            pl.BlockSpec((_H, _H), lambda i: (0, 0)),
        ],
        out_specs=[
            pl.BlockSpec((_BN, _H), lambda i: (i, 0)),
            pl.BlockSpec((_BN, _H), lambda i: (i, 0)),
        ],
        out_shape=[
            jax.ShapeDtypeStruct((_N, _H), jnp.float32),
            jax.ShapeDtypeStruct((_N, _H), jnp.float32),
        ],
    )(nodes, node_enc_W, node_enc_b.reshape(1, _H), Ws, Wr)

    SP, SQ = _sc_gather(P, Q, senders, receivers)

    a = bi_edges_indx[:, 0]
    bcol = bi_edges_indx[:, 1]
    pair_ids = jnp.arange(_E // 2, dtype=jnp.int32)
    win = jnp.full((_E,), -1, jnp.int32).at[a].set(pair_ids).at[bcol].set(pair_ids)
    touched = win >= 0
    Wacc = jnp.zeros((_E // 2,), jnp.float32).at[
        jnp.where(touched, win, 0)].add(jnp.where(touched, w, 0.0))
    G = jnp.zeros((_E,), jnp.float32).at[a].add(Wacc).at[bcol].add(Wacc)
    c = jnp.where(touched, 0.0, w) + 0.5 * G

    S0, sw = pl.pallas_call(
        _edge_body,
        grid=(_E // _BE2,),
        in_specs=[
            pl.BlockSpec((_BE2, 1), lambda i: (i, 0)),
            pl.BlockSpec((_BE2, _H), lambda i: (i, 0)),
            pl.BlockSpec((_BE2, _H), lambda i: (i, 0)),
            pl.BlockSpec((_BE2, 1), lambda i: (i, 0)),
            pl.BlockSpec((_BE2, 1), lambda i: (i, 0)),
            pl.BlockSpec((1, 1), lambda i: (0, 0)),
            pl.BlockSpec((1, _H), lambda i: (0, 0)),
            pl.BlockSpec((1, _H), lambda i: (0, 0)),
            pl.BlockSpec((_H, _H), lambda i: (0, 0)),
            pl.BlockSpec((1, _H), lambda i: (0, 0)),
        ],
        out_specs=[
            pl.BlockSpec((1, _H), lambda i: (0, 0)),
            pl.BlockSpec((1, 1), lambda i: (0, 0)),
        ],
        out_shape=[
            jax.ShapeDtypeStruct((1, _H), jnp.float32),
            jax.ShapeDtypeStruct((1, 1), jnp.float32),
        ],
    )(edges, SP, SQ, c.reshape(_E, 1), w.reshape(_E, 1), norm2,
      edge_enc_W, edge_enc_b.reshape(1, _H), We3, mp_edge_b.reshape(1, _H))

    alpha = (S0 @ edge_dec_W)[0] + edge_dec_b * sw[0, 0] + b

    diag_data = jnp.broadcast_to(alpha, (_N,))
    data = jnp.concatenate([tri_data, diag_data], axis=0)
    ar = jnp.arange(_N, dtype=jnp.int32)
    indices = jnp.concatenate(
        [jnp.stack([senders, receivers], axis=1), jnp.stack([ar, ar], axis=1)],
        axis=0)
    return data, indices


# speedup vs baseline: 4.9088x; 3.5922x over previous
import functools

import jax
import jax.numpy as jnp
from jax import lax
from jax.experimental import pallas as pl
from jax.experimental.pallas import tpu as pltpu
from jax.experimental.pallas import tpu_sc as plsc

_E = 800000
_N = 50000
_H = 16
_ER = 6250
_BE1 = 625
_BN = 2000
_BE2 = 4000


def _norm_tri_body(edges_ref, s_ref, r_ref, norm_ref, tri_ref):
    tri_ref[...] = jnp.where(s_ref[...] >= r_ref[...], edges_ref[...], 0.0)
    norm_ref[...] = jnp.max(jnp.abs(edges_ref[...])).reshape(1, 1)


def _pq_body(nodes_ref, nW_ref, nb_ref, Ws_ref, Wr_ref, p_ref, q_ref):
    n0 = jnp.tanh(nodes_ref[...] * nW_ref[...] + nb_ref[...])
    p_ref[...] = jnp.dot(n0, Ws_ref[...], preferred_element_type=jnp.float32)
    q_ref[...] = jnp.dot(n0, Wr_ref[...], preferred_element_type=jnp.float32)


def _edge_body(edges_ref, sp_ref, sq_ref, win_ref, g_ref, w_ref, norm_ref,
               eW_ref, eb_ref, We3_ref, be_ref, s0_ref, sw_ref):
    i = pl.program_id(0)
    e = edges_ref[...] / norm_ref[...]
    ef0 = jnp.tanh(e * eW_ref[...] + eb_ref[...])
    r = jnp.dot(ef0, We3_ref[...], preferred_element_type=jnp.float32) + be_ref[...]
    ef1 = jnp.tanh(sp_ref[...] + sq_ref[...] + r)
    c = jnp.where(win_ref[...] == 0, w_ref[...], 0.0) + 0.5 * g_ref[...]
    contrib = jnp.sum(c * ef1, axis=0, keepdims=True)
    wsum = jnp.sum(w_ref[...])

    @pl.when(i == 0)
    def _():
        s0_ref[...] = contrib
        sw_ref[...] = wsum.reshape(1, 1)

    @pl.when(i > 0)
    def _():
        s0_ref[...] = s0_ref[...] + contrib
        sw_ref[...] = sw_ref[...] + wsum.reshape(1, 1)


_NC = 2
_NS = 16
_NW = _NC * _NS
_EPW = _E // _NW
_GCH = 1000
_GIT = _EPW // _GCH


def _gather_body(p_hbm, q_hbm, s_hbm, r_hbm, sp_hbm, sq_hbm,
                 sidx_v, ridx_v, prow_v, qrow_v, sem1, sem2):
    wid = lax.axis_index("s") * _NC + lax.axis_index("c")
    base = wid * _EPW

    def step(t, carry):
        off = base + t * _GCH
        pltpu.sync_copy(s_hbm.at[pl.ds(off, _GCH)], sidx_v)
        pltpu.sync_copy(r_hbm.at[pl.ds(off, _GCH)], ridx_v)
        cp1 = pltpu.async_copy(p_hbm.at[sidx_v], prow_v, sem1)
        cp2 = pltpu.async_copy(q_hbm.at[ridx_v], qrow_v, sem2)
        cp1.wait()
        cp2.wait()
        pltpu.sync_copy(prow_v, sp_hbm.at[pl.ds(off, _GCH)])
        pltpu.sync_copy(qrow_v, sq_hbm.at[pl.ds(off, _GCH)])
        return carry

    lax.fori_loop(0, _GIT, step, 0)


def _sc_gather(P, Q, senders, receivers):
    mesh = plsc.VectorSubcoreMesh(core_axis_name="c", subcore_axis_name="s",
                                  num_cores=_NC, num_subcores=_NS)
    f = pl.kernel(
        _gather_body,
        out_type=[
            jax.ShapeDtypeStruct((_E, _H), jnp.float32),
            jax.ShapeDtypeStruct((_E, _H), jnp.float32),
        ],
        mesh=mesh,
        scratch_types=[
            pltpu.VMEM((_GCH,), jnp.int32),
            pltpu.VMEM((_GCH,), jnp.int32),
            pltpu.VMEM((_GCH, _H), jnp.float32),
            pltpu.VMEM((_GCH, _H), jnp.float32),
            pltpu.SemaphoreType.DMA,
            pltpu.SemaphoreType.DMA,
        ],
        compiler_params=pltpu.CompilerParams(use_tc_tiling_on_sc=False),
    )
    return f(P, Q, senders, receivers)


_EH = _E // 2
_PPS = _EH // _NS
_SPS = _E // _NS
_CHP = 1000
_CHS = 2000


def _coeff_body(a_hbm, b_hbm, w_hbm, pid_hbm, zi_hbm, zf_hbm,
                win_hbm, g_hbm,
                win_sh, g_sh, wacc_sh,
                ia_v, ib_v, v_v):
    sid = lax.axis_index("s")
    pbase = sid * _PPS
    sbase = sid * _SPS

    def init_step(k, carry):
        soff = sbase + k * _CHS
        pltpu.sync_copy(zi_hbm.at[pl.ds(soff, _CHS)], win_sh.at[pl.ds(soff, _CHS)])
        pltpu.sync_copy(zf_hbm.at[pl.ds(soff, _CHS)], g_sh.at[pl.ds(soff, _CHS)])
        return carry

    lax.fori_loop(0, _SPS // _CHS, init_step, 0)

    def initw_step(k, carry):
        poff = pbase + k * _CHP
        pltpu.sync_copy(zf_hbm.at[pl.ds(poff, _CHP)], wacc_sh.at[pl.ds(poff, _CHP)])
        return carry

    lax.fori_loop(0, _PPS // _CHP, initw_step, 0)

    @pl.when(sid == 0)
    def _():
        pltpu.sync_copy(zf_hbm.at[pl.ds(_EH, 8)], wacc_sh.at[pl.ds(_EH, 8)])

    plsc.subcore_barrier()

    def col0_step(k, carry):
        poff = pbase + k * _CHP
        ia = ia_v.at[pl.ds(0, _CHP)]
        ib = ib_v.at[pl.ds(0, _CHP)]
        pltpu.sync_copy(a_hbm.at[pl.ds(poff, _CHP)], ia)
        pltpu.sync_copy(pid_hbm.at[pl.ds(poff, _CHP)], ib)
        pltpu.sync_copy(ib, win_sh.at[ia])
        return carry

    lax.fori_loop(0, _PPS // _CHP, col0_step, 0)
    plsc.subcore_barrier()

    def col1_step(k, carry):
        poff = pbase + k * _CHP
        ia = ia_v.at[pl.ds(0, _CHP)]
        ib = ib_v.at[pl.ds(0, _CHP)]
        pltpu.sync_copy(b_hbm.at[pl.ds(poff, _CHP)], ia)
        pltpu.sync_copy(pid_hbm.at[pl.ds(poff, _CHP)], ib)
        pltpu.sync_copy(ib, win_sh.at[ia])
        return carry

    lax.fori_loop(0, _PPS // _CHP, col1_step, 0)
    plsc.subcore_barrier()

    def wacc_step(k, carry):
        soff = sbase + k * _CHS
        pltpu.sync_copy(win_sh.at[pl.ds(soff, _CHS)], ia_v)
        pltpu.sync_copy(w_hbm.at[pl.ds(soff, _CHS)], v_v)
        pltpu.sync_copy(v_v, wacc_sh.at[ia_v], add=True)
        return carry

    lax.fori_loop(0, _SPS // _CHS, wacc_step, 0)
    plsc.subcore_barrier()

    def g_step(k, carry):
        poff = pbase + k * _CHP
        ia = ia_v.at[pl.ds(0, _CHP)]
        ib = ib_v.at[pl.ds(0, _CHP)]
        vv = v_v.at[pl.ds(0, _CHP)]
        pltpu.sync_copy(wacc_sh.at[pl.ds(poff + 8, _CHP)], vv)
        pltpu.sync_copy(a_hbm.at[pl.ds(poff, _CHP)], ia)
        pltpu.sync_copy(b_hbm.at[pl.ds(poff, _CHP)], ib)
        pltpu.sync_copy(vv, g_sh.at[ia], add=True)
        pltpu.sync_copy(vv, g_sh.at[ib], add=True)
        return carry

    lax.fori_loop(0, _PPS // _CHP, g_step, 0)
    plsc.subcore_barrier()

    pltpu.sync_copy(win_sh.at[pl.ds(sbase, _SPS)], win_hbm.at[pl.ds(sbase, _SPS)])
    pltpu.sync_copy(g_sh.at[pl.ds(sbase, _SPS)], g_hbm.at[pl.ds(sbase, _SPS)])


def _sc_coeff(a, bcol, w, pid, zi, zf):
    mesh = plsc.VectorSubcoreMesh(core_axis_name="c", subcore_axis_name="s",
                                  num_cores=1, num_subcores=_NS)
    f = pl.kernel(
        _coeff_body,
        out_type=[
            jax.ShapeDtypeStruct((_E,), jnp.int32),
            jax.ShapeDtypeStruct((_E,), jnp.float32),
        ],
        mesh=mesh,
        scratch_types=[
            pltpu.VMEM_SHARED((_E,), jnp.int32),
            pltpu.VMEM_SHARED((_E,), jnp.float32),
            pltpu.VMEM_SHARED((_EH + 8,), jnp.float32),
            pltpu.VMEM((_CHS,), jnp.int32),
            pltpu.VMEM((_CHS,), jnp.int32),
            pltpu.VMEM((_CHS,), jnp.float32),
        ],
        compiler_params=pltpu.CompilerParams(use_tc_tiling_on_sc=False),
    )
    return f(a, bcol, w, pid, zi, zf)


def kernel(nodes, edges, receivers, senders, bi_edges_indx, lhs_nodes, lhs_edges,
           lhs_receivers, lhs_senders, node_enc_W, node_enc_b, edge_enc_W,
           edge_enc_b, mp_edge_W, mp_edge_b, mp_node_W, mp_node_b, edge_dec_W,
           edge_dec_b, w, b):
    edges1 = edges[:, 0]

    norm2, tri2 = pl.pallas_call(
        _norm_tri_body,
        out_shape=[
            jax.ShapeDtypeStruct((1, 1), jnp.float32),
            jax.ShapeDtypeStruct((_ER, 128), jnp.float32),
        ],
    )(edges1.reshape(_ER, 128), senders.reshape(_ER, 128),
      receivers.reshape(_ER, 128))
    tri_data = tri2.reshape(_E)

    Ws = mp_edge_W[0:_H]
    Wr = mp_edge_W[_H:2 * _H]
    We3 = mp_edge_W[2 * _H:3 * _H]
    P, Q = pl.pallas_call(
        _pq_body,
        grid=(_N // _BN,),
        in_specs=[
            pl.BlockSpec((_BN, 1), lambda i: (i, 0)),
            pl.BlockSpec((1, _H), lambda i: (0, 0)),
            pl.BlockSpec((1, _H), lambda i: (0, 0)),
            pl.BlockSpec((_H, _H), lambda i: (0, 0)),
            pl.BlockSpec((_H, _H), lambda i: (0, 0)),
        ],
        out_specs=[
            pl.BlockSpec((_BN, _H), lambda i: (i, 0)),
            pl.BlockSpec((_BN, _H), lambda i: (i, 0)),
        ],
        out_shape=[
            jax.ShapeDtypeStruct((_N, _H), jnp.float32),
            jax.ShapeDtypeStruct((_N, _H), jnp.float32),
        ],
    )(nodes, node_enc_W, node_enc_b.reshape(1, _H), Ws, Wr)

    SP, SQ = _sc_gather(P, Q, senders, receivers)

    a = bi_edges_indx[:, 0]
    bcol = bi_edges_indx[:, 1]
    pair_ids = jnp.arange(8, _EH + 8, dtype=jnp.int32)
    zi = jnp.zeros((_E,), jnp.int32)
    zf = jnp.zeros((_E,), jnp.float32)
    win, G = _sc_coeff(a, bcol, w, pair_ids, zi, zf)

    S0, sw = pl.pallas_call(
        _edge_body,
        grid=(_E // _BE2,),
        in_specs=[
            pl.BlockSpec((_BE2, 1), lambda i: (i, 0)),
            pl.BlockSpec((_BE2, _H), lambda i: (i, 0)),
            pl.BlockSpec((_BE2, _H), lambda i: (i, 0)),
            pl.BlockSpec((_BE2, 1), lambda i: (i, 0)),
            pl.BlockSpec((_BE2, 1), lambda i: (i, 0)),
            pl.BlockSpec((_BE2, 1), lambda i: (i, 0)),
            pl.BlockSpec((1, 1), lambda i: (0, 0)),
            pl.BlockSpec((1, _H), lambda i: (0, 0)),
            pl.BlockSpec((1, _H), lambda i: (0, 0)),
            pl.BlockSpec((_H, _H), lambda i: (0, 0)),
            pl.BlockSpec((1, _H), lambda i: (0, 0)),
        ],
        out_specs=[
            pl.BlockSpec((1, _H), lambda i: (0, 0)),
            pl.BlockSpec((1, 1), lambda i: (0, 0)),
        ],
        out_shape=[
            jax.ShapeDtypeStruct((1, _H), jnp.float32),
            jax.ShapeDtypeStruct((1, 1), jnp.float32),
        ],
    )(edges, SP, SQ, win.reshape(_E, 1), G.reshape(_E, 1), w.reshape(_E, 1), norm2,
      edge_enc_W, edge_enc_b.reshape(1, _H), We3, mp_edge_b.reshape(1, _H))

    alpha = (S0 @ edge_dec_W)[0] + edge_dec_b * sw[0, 0] + b

    diag_data = jnp.broadcast_to(alpha, (_N,))
    data = jnp.concatenate([tri_data, diag_data], axis=0)
    ar = jnp.arange(_N, dtype=jnp.int32)
    indices = jnp.concatenate(
        [jnp.stack([senders, receivers], axis=1), jnp.stack([ar, ar], axis=1)],
        axis=0)
    return data, indices


# speedup vs baseline: 9.5462x; 1.9447x over previous
import functools

import jax
import jax.numpy as jnp
from jax import lax
from jax.experimental import pallas as pl
from jax.experimental.pallas import tpu as pltpu
from jax.experimental.pallas import tpu_sc as plsc

_E = 800000
_N = 50000
_H = 16
_ER = 6250
_BE1 = 625
_BN = 2000
_BE2 = 8000


def _norm_tri_body(edges_ref, s_ref, r_ref, norm_ref, tri_ref):
    tri_ref[...] = jnp.where(s_ref[...] >= r_ref[...], edges_ref[...], 0.0)
    norm_ref[...] = jnp.max(jnp.abs(edges_ref[...])).reshape(1, 1)


def _pq_body(nodes_ref, nW_ref, nb_ref, Ws_ref, Wr_ref, p_ref, q_ref):
    n0 = jnp.tanh(nodes_ref[...] * nW_ref[...] + nb_ref[...])
    p_ref[...] = jnp.dot(n0, Ws_ref[...], preferred_element_type=jnp.float32)
    q_ref[...] = jnp.dot(n0, Wr_ref[...], preferred_element_type=jnp.float32)


def _cw_body(win_ref, g_ref, w_ref, c_ref, sw_ref):
    c_ref[...] = (jnp.where(win_ref[...] == 0, w_ref[...], 0.0)
                  + 0.5 * g_ref[...])
    sw_ref[...] = jnp.sum(w_ref[...]).reshape(1, 1)


def _edge_body(edges_ref, sp_ref, sq_ref, c_ref, norm_ref,
               eW_ref, eb_ref, We3_ref, be_ref, s0_ref):
    i = pl.program_id(0)
    e = edges_ref[...] / norm_ref[...]
    ef0 = jnp.tanh(e * eW_ref[...] + eb_ref[...])
    r = jnp.dot(ef0, We3_ref[...], preferred_element_type=jnp.float32) + be_ref[...]
    ef1 = jnp.tanh(sp_ref[...] + sq_ref[...] + r)
    crow = c_ref[...].reshape(1, _BE2)
    contrib = jnp.dot(crow, ef1, preferred_element_type=jnp.float32)

    @pl.when(i == 0)
    def _():
        s0_ref[...] = contrib

    @pl.when(i > 0)
    def _():
        s0_ref[...] = s0_ref[...] + contrib


_NC = 2
_NS = 16
_NW = _NC * _NS
_EPW = _E // _NW
_GCH = 1000
_GIT = _EPW // _GCH


def _gather_body(p_hbm, q_hbm, s_hbm, r_hbm, sp_hbm, sq_hbm,
                 sidx_v, ridx_v, prow_v, qrow_v, sem1, sem2):
    wid = lax.axis_index("s") * _NC + lax.axis_index("c")
    base = wid * _EPW

    def step(t, carry):
        off = base + t * _GCH
        pltpu.sync_copy(s_hbm.at[pl.ds(off, _GCH)], sidx_v)
        pltpu.sync_copy(r_hbm.at[pl.ds(off, _GCH)], ridx_v)
        cp1 = pltpu.async_copy(p_hbm.at[sidx_v], prow_v, sem1)
        cp2 = pltpu.async_copy(q_hbm.at[ridx_v], qrow_v, sem2)
        cp1.wait()
        cp2.wait()
        pltpu.sync_copy(prow_v, sp_hbm.at[pl.ds(off, _GCH)])
        pltpu.sync_copy(qrow_v, sq_hbm.at[pl.ds(off, _GCH)])
        return carry

    lax.fori_loop(0, _GIT, step, 0)


def _sc_gather(P, Q, senders, receivers):
    mesh = plsc.VectorSubcoreMesh(core_axis_name="c", subcore_axis_name="s",
                                  num_cores=_NC, num_subcores=_NS)
    f = pl.kernel(
        _gather_body,
        out_type=[
            jax.ShapeDtypeStruct((_E, _H), jnp.float32),
            jax.ShapeDtypeStruct((_E, _H), jnp.float32),
        ],
        mesh=mesh,
        scratch_types=[
            pltpu.VMEM((_GCH,), jnp.int32),
            pltpu.VMEM((_GCH,), jnp.int32),
            pltpu.VMEM((_GCH, _H), jnp.float32),
            pltpu.VMEM((_GCH, _H), jnp.float32),
            pltpu.SemaphoreType.DMA,
            pltpu.SemaphoreType.DMA,
        ],
        compiler_params=pltpu.CompilerParams(use_tc_tiling_on_sc=False),
    )
    return f(P, Q, senders, receivers)


_EH = _E // 2
_PPS = _EH // _NS
_SPS = _E // _NS
_CHP = 1000
_CHS = 2000


def _coeff_body(a_hbm, b_hbm, w_hbm, pid_hbm, zi_hbm, zf_hbm,
                win_hbm, g_hbm,
                win_sh, g_sh, wacc_sh,
                ia_v, ib_v, v_v):
    sid = lax.axis_index("s")
    pbase = sid * _PPS
    sbase = sid * _SPS

    def init_step(k, carry):
        soff = sbase + k * _CHS
        pltpu.sync_copy(zi_hbm.at[pl.ds(soff, _CHS)], win_sh.at[pl.ds(soff, _CHS)])
        pltpu.sync_copy(zf_hbm.at[pl.ds(soff, _CHS)], g_sh.at[pl.ds(soff, _CHS)])
        return carry

    lax.fori_loop(0, _SPS // _CHS, init_step, 0)

    def initw_step(k, carry):
        poff = pbase + k * _CHP
        pltpu.sync_copy(zf_hbm.at[pl.ds(poff, _CHP)], wacc_sh.at[pl.ds(poff, _CHP)])
        return carry

    lax.fori_loop(0, _PPS // _CHP, initw_step, 0)

    @pl.when(sid == 0)
    def _():
        pltpu.sync_copy(zf_hbm.at[pl.ds(_EH, 8)], wacc_sh.at[pl.ds(_EH, 8)])

    plsc.subcore_barrier()

    def col0_step(k, carry):
        poff = pbase + k * _CHP
        ia = ia_v.at[pl.ds(0, _CHP)]
        ib = ib_v.at[pl.ds(0, _CHP)]
        pltpu.sync_copy(a_hbm.at[pl.ds(poff, _CHP)], ia)
        pltpu.sync_copy(pid_hbm.at[pl.ds(poff, _CHP)], ib)
        pltpu.sync_copy(ib, win_sh.at[ia])
        return carry

    lax.fori_loop(0, _PPS // _CHP, col0_step, 0)
    plsc.subcore_barrier()

    def col1_step(k, carry):
        poff = pbase + k * _CHP
        ia = ia_v.at[pl.ds(0, _CHP)]
        ib = ib_v.at[pl.ds(0, _CHP)]
        pltpu.sync_copy(b_hbm.at[pl.ds(poff, _CHP)], ia)
        pltpu.sync_copy(pid_hbm.at[pl.ds(poff, _CHP)], ib)
        pltpu.sync_copy(ib, win_sh.at[ia])
        return carry

    lax.fori_loop(0, _PPS // _CHP, col1_step, 0)
    plsc.subcore_barrier()

    def wacc_step(k, carry):
        soff = sbase + k * _CHS
        pltpu.sync_copy(win_sh.at[pl.ds(soff, _CHS)], ia_v)
        pltpu.sync_copy(w_hbm.at[pl.ds(soff, _CHS)], v_v)
        pltpu.sync_copy(v_v, wacc_sh.at[ia_v], add=True)
        return carry

    lax.fori_loop(0, _SPS // _CHS, wacc_step, 0)
    plsc.subcore_barrier()

    def g_step(k, carry):
        poff = pbase + k * _CHP
        ia = ia_v.at[pl.ds(0, _CHP)]
        ib = ib_v.at[pl.ds(0, _CHP)]
        vv = v_v.at[pl.ds(0, _CHP)]
        pltpu.sync_copy(wacc_sh.at[pl.ds(poff + 8, _CHP)], vv)
        pltpu.sync_copy(a_hbm.at[pl.ds(poff, _CHP)], ia)
        pltpu.sync_copy(b_hbm.at[pl.ds(poff, _CHP)], ib)
        pltpu.sync_copy(vv, g_sh.at[ia], add=True)
        pltpu.sync_copy(vv, g_sh.at[ib], add=True)
        return carry

    lax.fori_loop(0, _PPS // _CHP, g_step, 0)
    plsc.subcore_barrier()

    pltpu.sync_copy(win_sh.at[pl.ds(sbase, _SPS)], win_hbm.at[pl.ds(sbase, _SPS)])
    pltpu.sync_copy(g_sh.at[pl.ds(sbase, _SPS)], g_hbm.at[pl.ds(sbase, _SPS)])


def _sc_coeff(a, bcol, w, pid, zi, zf):
    mesh = plsc.VectorSubcoreMesh(core_axis_name="c", subcore_axis_name="s",
                                  num_cores=1, num_subcores=_NS)
    f = pl.kernel(
        _coeff_body,
        out_type=[
            jax.ShapeDtypeStruct((_E,), jnp.int32),
            jax.ShapeDtypeStruct((_E,), jnp.float32),
        ],
        mesh=mesh,
        scratch_types=[
            pltpu.VMEM_SHARED((_E,), jnp.int32),
            pltpu.VMEM_SHARED((_E,), jnp.float32),
            pltpu.VMEM_SHARED((_EH + 8,), jnp.float32),
            pltpu.VMEM((_CHS,), jnp.int32),
            pltpu.VMEM((_CHS,), jnp.int32),
            pltpu.VMEM((_CHS,), jnp.float32),
        ],
        compiler_params=pltpu.CompilerParams(use_tc_tiling_on_sc=False),
    )
    return f(a, bcol, w, pid, zi, zf)


def kernel(nodes, edges, receivers, senders, bi_edges_indx, lhs_nodes, lhs_edges,
           lhs_receivers, lhs_senders, node_enc_W, node_enc_b, edge_enc_W,
           edge_enc_b, mp_edge_W, mp_edge_b, mp_node_W, mp_node_b, edge_dec_W,
           edge_dec_b, w, b):
    edges1 = edges[:, 0]

    norm2, tri2 = pl.pallas_call(
        _norm_tri_body,
        out_shape=[
            jax.ShapeDtypeStruct((1, 1), jnp.float32),
            jax.ShapeDtypeStruct((_ER, 128), jnp.float32),
        ],
    )(edges1.reshape(_ER, 128), senders.reshape(_ER, 128),
      receivers.reshape(_ER, 128))
    tri_data = tri2.reshape(_E)

    Ws = mp_edge_W[0:_H]
    Wr = mp_edge_W[_H:2 * _H]
    We3 = mp_edge_W[2 * _H:3 * _H]
    P, Q = pl.pallas_call(
        _pq_body,
        grid=(_N // _BN,),
        in_specs=[
            pl.BlockSpec((_BN, 1), lambda i: (i, 0)),
            pl.BlockSpec((1, _H), lambda i: (0, 0)),
            pl.BlockSpec((1, _H), lambda i: (0, 0)),
            pl.BlockSpec((_H, _H), lambda i: (0, 0)),
            pl.BlockSpec((_H, _H), lambda i: (0, 0)),
        ],
        out_specs=[
            pl.BlockSpec((_BN, _H), lambda i: (i, 0)),
            pl.BlockSpec((_BN, _H), lambda i: (i, 0)),
        ],
        out_shape=[
            jax.ShapeDtypeStruct((_N, _H), jnp.float32),
            jax.ShapeDtypeStruct((_N, _H), jnp.float32),
        ],
    )(nodes, node_enc_W, node_enc_b.reshape(1, _H), Ws, Wr)

    SP, SQ = _sc_gather(P, Q, senders, receivers)

    a = bi_edges_indx[:, 0]
    bcol = bi_edges_indx[:, 1]
    pair_ids = jnp.arange(8, _EH + 8, dtype=jnp.int32)
    zi = jnp.zeros((_E,), jnp.int32)
    zf = jnp.zeros((_E,), jnp.float32)
    win, G = _sc_coeff(a, bcol, w, pair_ids, zi, zf)

    c2d, sw = pl.pallas_call(
        _cw_body,
        out_shape=[
            jax.ShapeDtypeStruct((_ER, 128), jnp.float32),
            jax.ShapeDtypeStruct((1, 1), jnp.float32),
        ],
    )(win.reshape(_ER, 128), G.reshape(_ER, 128), w.reshape(_ER, 128))

    S0 = pl.pallas_call(
        _edge_body,
        grid=(_E // _BE2,),
        in_specs=[
            pl.BlockSpec((_BE2, 1), lambda i: (i, 0)),
            pl.BlockSpec((_BE2, _H), lambda i: (i, 0)),
            pl.BlockSpec((_BE2, _H), lambda i: (i, 0)),
            pl.BlockSpec((1, 1, _BE2), lambda i: (i, 0, 0)),
            pl.BlockSpec((1, 1), lambda i: (0, 0)),
            pl.BlockSpec((1, _H), lambda i: (0, 0)),
            pl.BlockSpec((1, _H), lambda i: (0, 0)),
            pl.BlockSpec((_H, _H), lambda i: (0, 0)),
            pl.BlockSpec((1, _H), lambda i: (0, 0)),
        ],
        out_specs=pl.BlockSpec((1, _H), lambda i: (0, 0)),
        out_shape=jax.ShapeDtypeStruct((1, _H), jnp.float32),
    )(edges, SP, SQ, c2d.reshape(_E // _BE2, 1, _BE2), norm2,
      edge_enc_W, edge_enc_b.reshape(1, _H), We3, mp_edge_b.reshape(1, _H))

    alpha = (S0 @ edge_dec_W)[0] + edge_dec_b * sw[0, 0] + b

    diag_data = jnp.broadcast_to(alpha, (_N,))
    data = jnp.concatenate([tri_data, diag_data], axis=0)
    ar = jnp.arange(_N, dtype=jnp.int32)
    indices = jnp.concatenate(
        [jnp.stack([senders, receivers], axis=1), jnp.stack([ar, ar], axis=1)],
        axis=0)
    return data, indices
